# baseline (speedup 1.0000x reference)
import functools
import jax
import jax.numpy as jnp
from jax import lax
from jax.experimental import pallas as pl
from jax.experimental.pallas import tpu as pltpu
from jax.experimental.pallas import tpu_sc as plsc

_NLANE = 16


def _sc_body(x_hbm, wt_hbm, pe_hbm, sp_hbm, out_hbm, wt_v, pe_v, sp_v, x_v,
             out_v0, out_v1, sem0, sem1):
    n_sp = sp_v.shape[0]
    d_model = out_v0.shape[1]
    ndc = d_model // _NLANE
    nc = 2
    wid = lax.axis_index("s") * nc + lax.axis_index("c")
    b_per_w = x_hbm.shape[0] // 32

    pltpu.sync_copy(wt_hbm, wt_v)
    pltpu.sync_copy(pe_hbm, pe_v)
    pltpu.sync_copy(sp_hbm, sp_v)

    w = [[wt_v[k, pl.ds(dc * _NLANE, _NLANE)] for k in range(4)] for dc in range(ndc)]

    def make_tg_body(c0, out_v):
        def tg_body(tg):
            r0 = tg * n_sp
            tok0 = c0 + r0
            t = tok0 // n_sp
            pec = [pe_v[t, pl.ds(dc * _NLANE, _NLANE)] for dc in range(ndc)]
            for s in range(n_sp):
                r = r0 + s
                xb = 3 * (tok0 + s)
                xv = x_v[0, pl.ds(xb, _NLANE)]
                x0s = xv[0]
                x1s = xv[1]
                x2s = xv[2]
                m0 = x0s != x0s
                m1 = x1s != x1s
                m2 = x2s != x2s
                x0s = jnp.where(m0, 0.0, x0s)
                x1s = jnp.where(m1, 0.0, x1s)
                x2s = jnp.where(m2, 0.0, x2s)
                mfs = jnp.where(m0 | m1 | m2, 1.0, 0.0)
                x0 = jnp.full((_NLANE,), x0s, jnp.float32)
                x1 = jnp.full((_NLANE,), x1s, jnp.float32)
                x2 = jnp.full((_NLANE,), x2s, jnp.float32)
                mf = jnp.full((_NLANE,), mfs, jnp.float32)
                for dc in range(ndc):
                    acc = pec[dc] + sp_v[s, pl.ds(dc * _NLANE, _NLANE)]
                    acc = acc + x0 * w[dc][0] + x1 * w[dc][1]
                    acc = acc + x2 * w[dc][2] + mf * w[dc][3]
                    out_v[r, pl.ds(dc * _NLANE, _NLANE)] = acc
        return tg_body

    def batch_body(bi, _):
        b = wid * b_per_w + bi
        pltpu.sync_copy(x_hbm.at[b], x_v)
        bufs = (out_v0, out_v1)
        sems = (sem0, sem1)
        copies = [None, None]
        for ci, (c0, csz) in enumerate(((0, 400), (400, 400), (800, 400), (1200, 50))):
            out_v = bufs[ci % 2]
            if copies[ci % 2] is not None:
                copies[ci % 2].wait()
            plsc.parallel_loop(0, csz // n_sp, unroll=1)(make_tg_body(c0, out_v))
            cp = pltpu.make_async_copy(out_v.at[pl.ds(0, csz)],
                                       out_hbm.at[b, pl.ds(c0, csz)],
                                       sems[ci % 2])
            cp.start()
            copies[ci % 2] = cp
        copies[0].wait()
        copies[1].wait()
        return 0

    lax.fori_loop(0, b_per_w, batch_body, 0)


def kernel(x, W, b, space_table, nan_table, pe):
    bsize = x.shape[0]
    d_x = W.shape[1]
    d_model = W.shape[0]
    xr = x.reshape(bsize, -1, d_x)
    ntok = xr.shape[1]
    xflat = x.reshape(bsize, ntok * d_x)
    xf = jnp.pad(xflat, ((0, 0), (0, _NLANE))).reshape(bsize, 1, ntok * d_x + _NLANE)

    wt4 = jnp.concatenate([W.T, (nan_table[1] - nan_table[0])[None, :]], axis=0)
    pe_eff = pe + (b + nan_table[0])[None, :]

    mesh = plsc.VectorSubcoreMesh(core_axis_name="c", subcore_axis_name="s")
    sc_fn = functools.partial(
        pl.kernel,
        mesh=mesh,
        out_type=jax.ShapeDtypeStruct((bsize, ntok, d_model), jnp.float32),
        scratch_types=[
            pltpu.VMEM(wt4.shape, jnp.float32),
            pltpu.VMEM(pe_eff.shape, jnp.float32),
            pltpu.VMEM(space_table.shape, jnp.float32),
            pltpu.VMEM((1, ntok * d_x + _NLANE), jnp.float32),
            pltpu.VMEM((400, d_model), jnp.float32),
            pltpu.VMEM((400, d_model), jnp.float32),
            pltpu.SemaphoreType.DMA,
            pltpu.SemaphoreType.DMA,
        ],
    )(_sc_body)
    return sc_fn(xf, wt4, pe_eff, space_table)


# speedup vs baseline: 3.3957x; 3.3957x over previous
import jax
import jax.numpy as jnp
from jax.experimental import pallas as pl
from jax.experimental.pallas import tpu as pltpu


def _body(x_ref, wt4_ref, bn_ref, pe_ref, sp_ref, out_ref, base_ref):
    ntok, d_model = base_ref.shape
    t_steps = pe_ref.shape[0]
    n_sp = sp_ref.shape[0]
    bb = x_ref.shape[0]

    @pl.when(pl.program_id(0) == 0)
    def _():
        ri = jax.lax.broadcasted_iota(jnp.int32, (ntok, t_steps), 0) // n_sp
        ci = jax.lax.broadcasted_iota(jnp.int32, (ntok, t_steps), 1)
        rt = (ri == ci).astype(jnp.float32)
        si = jax.lax.broadcasted_iota(jnp.int32, (ntok, n_sp), 0) % n_sp
        cj = jax.lax.broadcasted_iota(jnp.int32, (ntok, n_sp), 1)
        rs = (si == cj).astype(jnp.float32)
        base = jnp.dot(rt, pe_ref[...], preferred_element_type=jnp.float32)
        base = base + jnp.dot(rs, sp_ref[...], preferred_element_type=jnp.float32)
        base_ref[...] = base + bn_ref[...]

    xb = x_ref[...]
    m3 = jnp.isnan(xb)
    xc = jnp.where(m3, 0.0, xb)
    maskf = jnp.max(m3.astype(jnp.float32), axis=-1, keepdims=True)
    xin = jnp.concatenate([xc, maskf], axis=-1)
    base = base_ref[...]
    wt4 = wt4_ref[...]
    for i in range(bb):
        out_ref[i] = jnp.dot(xin[i], wt4, preferred_element_type=jnp.float32) + base


def kernel(x, W, b, space_table, nan_table, pe):
    bsize = x.shape[0]
    d_x = W.shape[1]
    d_model = W.shape[0]
    xr = x.reshape(bsize, -1, d_x)
    ntok = xr.shape[1]

    wt4 = jnp.concatenate([W.T, (nan_table[1] - nan_table[0])[None, :]], axis=0)
    bn = (b + nan_table[0]).reshape(1, -1)

    bb = 16
    grid = (bsize // bb,)
    out = pl.pallas_call(
        _body,
        grid=grid,
        in_specs=[
            pl.BlockSpec((bb, ntok, d_x), lambda i: (i, 0, 0)),
            pl.BlockSpec((d_x + 1, d_model), lambda i: (0, 0)),
            pl.BlockSpec((1, d_model), lambda i: (0, 0)),
            pl.BlockSpec(pe.shape, lambda i: (0, 0)),
            pl.BlockSpec(space_table.shape, lambda i: (0, 0)),
        ],
        out_specs=pl.BlockSpec((bb, ntok, d_model), lambda i: (i, 0, 0)),
        out_shape=jax.ShapeDtypeStruct((bsize, ntok, d_model), jnp.float32),
        scratch_shapes=[pltpu.VMEM((ntok, d_model), jnp.float32)],
    )(xr, wt4, bn, pe, space_table)
    return out
